# baseline (speedup 1.0000x reference)
import jax
import jax.numpy as jnp
from jax.experimental import pallas as pl
from jax.experimental.pallas import tpu as pltpu

_BLOCK_ROWS = 2000


def _mlp_body(x_ref, w1_ref, b1_ref, w2_ref, b2_ref, out_ref):
    h = jnp.dot(
        x_ref[...].astype(jnp.bfloat16),
        w1_ref[...].astype(jnp.bfloat16),
        preferred_element_type=jnp.float32,
    ).astype(jnp.bfloat16)
    h = jnp.maximum(h + b1_ref[...].astype(jnp.bfloat16), jnp.bfloat16(0))
    out = jnp.dot(
        h,
        w2_ref[...].astype(jnp.bfloat16),
        preferred_element_type=jnp.float32,
    )
    out_ref[...] = out + b2_ref[...]


def kernel(x, edge_index, W1, b1, W2, b2):
    n, d_in = x.shape
    d_hid = W1.shape[1]
    d_out = W2.shape[1]
    block = _BLOCK_ROWS if n % _BLOCK_ROWS == 0 else n
    grid = (n // block,)
    out = pl.pallas_call(
        _mlp_body,
        grid=grid,
        in_specs=[
            pl.BlockSpec((block, d_in), lambda i: (i, 0)),
            pl.BlockSpec((d_in, d_hid), lambda i: (0, 0)),
            pl.BlockSpec((d_hid,), lambda i: (0,)),
            pl.BlockSpec((d_hid, d_out), lambda i: (0, 0)),
            pl.BlockSpec((d_out,), lambda i: (0,)),
        ],
        out_specs=pl.BlockSpec((block, d_out), lambda i: (i, 0)),
        out_shape=jax.ShapeDtypeStruct((n, d_out), jnp.float32),
        compiler_params=pltpu.CompilerParams(
            dimension_semantics=("parallel",),
        ),
    )(x, W1, b1, W2, b2)
    return (x, out)


# speedup vs baseline: 1.0248x; 1.0248x over previous
import jax
import jax.numpy as jnp
from jax.experimental import pallas as pl
from jax.experimental.pallas import tpu as pltpu

_N = 10000
_CHUNK = 2000
_NCHUNK = _N // _CHUNK


def _mlp_body(x_hbm, w1_ref, b1_ref, w2_ref, b2_ref, out_hbm,
              x_vmem, out_vmem, in_sems, out_sems):
    for c in range(_NCHUNK):
        pltpu.make_async_copy(
            x_hbm.at[pl.ds(c * _CHUNK, _CHUNK), :],
            x_vmem.at[pl.ds(c * _CHUNK, _CHUNK), :],
            in_sems.at[c],
        ).start()
    for c in range(_NCHUNK):
        pltpu.make_async_copy(
            x_hbm.at[pl.ds(c * _CHUNK, _CHUNK), :],
            x_vmem.at[pl.ds(c * _CHUNK, _CHUNK), :],
            in_sems.at[c],
        ).wait()
        xb = x_vmem[pl.ds(c * _CHUNK, _CHUNK), :].astype(jnp.bfloat16)
        h = jnp.dot(
            xb, w1_ref[...].astype(jnp.bfloat16),
            preferred_element_type=jnp.float32,
        ).astype(jnp.bfloat16)
        h = jnp.maximum(h + b1_ref[...].astype(jnp.bfloat16), jnp.bfloat16(0))
        out = jnp.dot(
            h, w2_ref[...].astype(jnp.bfloat16),
            preferred_element_type=jnp.float32,
        )
        out_vmem[pl.ds(c * _CHUNK, _CHUNK), :] = out + b2_ref[...]
        pltpu.make_async_copy(
            out_vmem.at[pl.ds(c * _CHUNK, _CHUNK), :],
            out_hbm.at[pl.ds(c * _CHUNK, _CHUNK), :],
            out_sems.at[c],
        ).start()
    for c in range(_NCHUNK):
        pltpu.make_async_copy(
            out_vmem.at[pl.ds(c * _CHUNK, _CHUNK), :],
            out_hbm.at[pl.ds(c * _CHUNK, _CHUNK), :],
            out_sems.at[c],
        ).wait()


def kernel(x, edge_index, W1, b1, W2, b2):
    n, d_in = x.shape
    d_hid = W1.shape[1]
    d_out = W2.shape[1]
    out = pl.pallas_call(
        _mlp_body,
        in_specs=[
            pl.BlockSpec(memory_space=pltpu.HBM),
            pl.BlockSpec(memory_space=pltpu.VMEM),
            pl.BlockSpec(memory_space=pltpu.VMEM),
            pl.BlockSpec(memory_space=pltpu.VMEM),
            pl.BlockSpec(memory_space=pltpu.VMEM),
        ],
        out_specs=pl.BlockSpec(memory_space=pltpu.HBM),
        out_shape=jax.ShapeDtypeStruct((n, d_out), jnp.float32),
        scratch_shapes=[
            pltpu.VMEM((n, d_in), jnp.float32),
            pltpu.VMEM((n, d_out), jnp.float32),
            pltpu.SemaphoreType.DMA((_NCHUNK,)),
            pltpu.SemaphoreType.DMA((_NCHUNK,)),
        ],
    )(x, W1, b1, W2, b2)
    return (x, out)


# speedup vs baseline: 1.1738x; 1.1453x over previous
import jax
import jax.numpy as jnp
from jax.experimental import pallas as pl
from jax.experimental.pallas import tpu as pltpu

_BLOCK_ROWS = 5000


def _mlp_body(x_ref, w1_ref, b1_ref, w2_ref, b2_ref, out_ref):
    h = jnp.dot(
        x_ref[...].astype(jnp.bfloat16),
        w1_ref[...].astype(jnp.bfloat16),
        preferred_element_type=jnp.float32,
    ).astype(jnp.bfloat16)
    h = jnp.maximum(h + b1_ref[...].astype(jnp.bfloat16), jnp.bfloat16(0))
    out = jnp.dot(
        h,
        w2_ref[...].astype(jnp.bfloat16),
        preferred_element_type=jnp.float32,
    )
    out_ref[...] = out + b2_ref[...]


def kernel(x, edge_index, W1, b1, W2, b2):
    n, d_in = x.shape
    d_hid = W1.shape[1]
    d_out = W2.shape[1]
    block = _BLOCK_ROWS if n % _BLOCK_ROWS == 0 else n
    grid = (n // block,)
    out = pl.pallas_call(
        _mlp_body,
        grid=grid,
        in_specs=[
            pl.BlockSpec((block, d_in), lambda i: (i, 0)),
            pl.BlockSpec((d_in, d_hid), lambda i: (0, 0)),
            pl.BlockSpec((d_hid,), lambda i: (0,)),
            pl.BlockSpec((d_hid, d_out), lambda i: (0, 0)),
            pl.BlockSpec((d_out,), lambda i: (0,)),
        ],
        out_specs=pl.BlockSpec((block, d_out), lambda i: (i, 0)),
        out_shape=jax.ShapeDtypeStruct((n, d_out), jnp.float32),
        compiler_params=pltpu.CompilerParams(
            dimension_semantics=("parallel",),
        ),
    )(x, W1, b1, W2, b2)
    return (x, out)
